# baseline (speedup 1.0000x reference)
import functools
import jax
import jax.numpy as jnp
from jax import lax
from jax.experimental import pallas as pl
from jax.experimental.pallas import tpu as pltpu
from jax.experimental.pallas import tpu_sc as plsc

N = 65536
K = 1024
NC = 1
NS = 16
L = 16
NW = NC * NS
CHUNK = N // NW
GROUPS = CHUNK // L

_HALVES = (512, 256, 128, 64, 32, 16, 8, 4, 2, 1)

_mesh = plsc.VectorSubcoreMesh(
    core_axis_name="c", subcore_axis_name="s", num_cores=NC
)


@functools.partial(
    pl.kernel,
    mesh=_mesh,
    out_type=jax.ShapeDtypeStruct((N,), jnp.int32),
    scratch_types=[
        pltpu.VMEM((K,), jnp.float32),
        pltpu.VMEM((CHUNK,), jnp.float32),
        pltpu.VMEM((CHUNK,), jnp.int32),
    ],
    compiler_params=pltpu.CompilerParams(needs_layout_passes=False),
)
def _tokenize(value_hbm, embed_hbm, out_hbm, embed_v, vals_v, out_v):
    wid = lax.axis_index("s") * NC + lax.axis_index("c")
    base = wid * CHUNK
    pltpu.sync_copy(embed_hbm, embed_v)
    pltpu.sync_copy(value_hbm.at[pl.ds(base, CHUNK)], vals_v)

    @plsc.parallel_loop(0, GROUPS, unroll=8)
    def group(g):
        v = vals_v[pl.ds(g * L, L)]
        c = jnp.zeros((L,), jnp.int32)
        for half in _HALVES:
            ev = plsc.load_gather(embed_v, [c + (half - 1)])
            c = c + jnp.where(ev < v, half, 0)
        ec = plsc.load_gather(embed_v, [c])
        i0 = c + jnp.where(ec < v, 1, 0)
        ea = plsc.load_gather(embed_v, [jnp.maximum(i0 - 1, 0)])
        eb = plsc.load_gather(embed_v, [jnp.minimum(i0, K - 1)])
        ind = ((v - ea) > (eb - v)) & (i0 < K)
        dstar = jnp.where(ind, eb - v, v - ea)
        c2 = jnp.zeros((L,), jnp.int32)
        for half in _HALVES:
            ev = plsc.load_gather(embed_v, [c2 + (half - 1)])
            c2 = c2 + jnp.where((v - ev) > dstar, half, 0)
        ec2 = plsc.load_gather(embed_v, [c2])
        ans = c2 + jnp.where((v - ec2) > dstar, 1, 0)
        out_v[pl.ds(g * L, L)] = ans

    pltpu.sync_copy(out_v, out_hbm.at[pl.ds(base, CHUNK)])


def kernel(value, embed):
    idx = _tokenize(value, embed)
    return idx[:, None]


# speedup vs baseline: 1.3696x; 1.3696x over previous
import functools
import jax
import jax.numpy as jnp
from jax import lax
from jax.experimental import pallas as pl
from jax.experimental.pallas import tpu as pltpu
from jax.experimental.pallas import tpu_sc as plsc

N = 65536
K = 1024
NC = 2
NS = 16
L = 16
NW = NC * NS
CHUNK = N // NW
GROUPS = CHUNK // L

_HALVES = (512, 256, 128, 64, 32, 16, 8, 4, 2, 1)

_mesh = plsc.VectorSubcoreMesh(
    core_axis_name="c", subcore_axis_name="s", num_cores=NC
)


@functools.partial(
    pl.kernel,
    mesh=_mesh,
    out_type=jax.ShapeDtypeStruct((N,), jnp.int32),
    scratch_types=[
        pltpu.VMEM((K,), jnp.float32),
        pltpu.VMEM((CHUNK,), jnp.float32),
        pltpu.VMEM((CHUNK,), jnp.int32),
    ],
    compiler_params=pltpu.CompilerParams(needs_layout_passes=False),
)
def _tokenize(value_hbm, embed_hbm, out_hbm, embed_v, vals_v, out_v):
    wid = lax.axis_index("s") * NC + lax.axis_index("c")
    base = wid * CHUNK
    pltpu.sync_copy(embed_hbm, embed_v)
    pltpu.sync_copy(value_hbm.at[pl.ds(base, CHUNK)], vals_v)

    @plsc.parallel_loop(0, GROUPS, unroll=16)
    def group(g):
        v = vals_v[pl.ds(g * L, L)]
        c = jnp.zeros((L,), jnp.int32)
        for half in _HALVES:
            ev = plsc.load_gather(embed_v, [c + (half - 1)])
            c = c + jnp.where(ev < v, half, 0)
        ec = plsc.load_gather(embed_v, [c])
        i0 = c + jnp.where(ec < v, 1, 0)
        ea = plsc.load_gather(embed_v, [jnp.maximum(i0 - 1, 0)])
        eb = plsc.load_gather(embed_v, [jnp.minimum(i0, K - 1)])
        ind = ((v - ea) > (eb - v)) & (i0 < K)
        dstar = jnp.where(ind, eb - v, v - ea)
        c2 = jnp.zeros((L,), jnp.int32)
        for half in _HALVES:
            ev = plsc.load_gather(embed_v, [c2 + (half - 1)])
            c2 = c2 + jnp.where((v - ev) > dstar, half, 0)
        ec2 = plsc.load_gather(embed_v, [c2])
        ans = c2 + jnp.where((v - ec2) > dstar, 1, 0)
        out_v[pl.ds(g * L, L)] = ans

    pltpu.sync_copy(out_v, out_hbm.at[pl.ds(base, CHUNK)])


def kernel(value, embed):
    idx = _tokenize(value, embed)
    return idx[:, None]


# speedup vs baseline: 1.3715x; 1.0014x over previous
import functools
import jax
import jax.numpy as jnp
from jax import lax
from jax.experimental import pallas as pl
from jax.experimental.pallas import tpu as pltpu
from jax.experimental.pallas import tpu_sc as plsc

N = 65536
K = 1024
NC = 2
NS = 16
L = 16
NW = NC * NS
CHUNK = N // NW
GROUPS = CHUNK // L

_HALVES = (512, 256, 128, 64, 32, 16, 8, 4, 2, 1)

_mesh = plsc.VectorSubcoreMesh(
    core_axis_name="c", subcore_axis_name="s", num_cores=NC
)


@functools.partial(
    pl.kernel,
    mesh=_mesh,
    out_type=jax.ShapeDtypeStruct((N,), jnp.int32),
    scratch_types=[
        pltpu.VMEM((K,), jnp.float32),
        pltpu.VMEM((CHUNK,), jnp.float32),
        pltpu.VMEM((CHUNK,), jnp.int32),
    ],
    compiler_params=pltpu.CompilerParams(needs_layout_passes=False),
)
def _tokenize(value_hbm, embed_hbm, out_hbm, embed_v, vals_v, out_v):
    wid = lax.axis_index("s") * NC + lax.axis_index("c")
    base = wid * CHUNK
    with jax.named_scope("dma_in"):
        pltpu.sync_copy(embed_hbm, embed_v)
        pltpu.sync_copy(value_hbm.at[pl.ds(base, CHUNK)], vals_v)

    sscope = jax.named_scope("search")
    sscope.__enter__()

    @plsc.parallel_loop(0, GROUPS, unroll=16)
    def group(g):
        v = vals_v[pl.ds(g * L, L)]
        c = jnp.zeros((L,), jnp.int32)
        for half in _HALVES:
            ev = plsc.load_gather(embed_v, [c + (half - 1)])
            c = c + jnp.where(ev < v, half, 0)
        ec = plsc.load_gather(embed_v, [c])
        i0 = c + jnp.where(ec < v, 1, 0)
        ea = plsc.load_gather(embed_v, [jnp.maximum(i0 - 1, 0)])
        eb = plsc.load_gather(embed_v, [jnp.minimum(i0, K - 1)])
        ind = ((v - ea) > (eb - v)) & (i0 < K)
        dstar = jnp.where(ind, eb - v, v - ea)
        c2 = jnp.zeros((L,), jnp.int32)
        for half in _HALVES:
            ev = plsc.load_gather(embed_v, [c2 + (half - 1)])
            c2 = c2 + jnp.where((v - ev) > dstar, half, 0)
        ec2 = plsc.load_gather(embed_v, [c2])
        ans = c2 + jnp.where((v - ec2) > dstar, 1, 0)
        out_v[pl.ds(g * L, L)] = ans

    sscope.__exit__(None, None, None)
    with jax.named_scope("dma_out"):
        pltpu.sync_copy(out_v, out_hbm.at[pl.ds(base, CHUNK)])


def kernel(value, embed):
    idx = _tokenize(value, embed)
    return idx[:, None]


# speedup vs baseline: 1.8223x; 1.3287x over previous
import functools
import jax
import jax.numpy as jnp
from jax import lax
from jax.experimental import pallas as pl
from jax.experimental.pallas import tpu as pltpu
from jax.experimental.pallas import tpu_sc as plsc

N = 65536
K = 1024
NC = 2
NS = 16
L = 16
NW = NC * NS
CHUNK = N // NW
GROUPS = CHUNK // L

_HALVES = (512, 256, 128, 64, 32, 16, 8, 4, 2, 1)

_mesh = plsc.VectorSubcoreMesh(
    core_axis_name="c", subcore_axis_name="s", num_cores=NC
)


@functools.partial(
    pl.kernel,
    mesh=_mesh,
    out_type=jax.ShapeDtypeStruct((N,), jnp.int32),
    scratch_types=[
        pltpu.VMEM((K * L,), jnp.float32),
        pltpu.VMEM((CHUNK,), jnp.float32),
        pltpu.VMEM((CHUNK,), jnp.int32),
    ],
    compiler_params=pltpu.CompilerParams(needs_layout_passes=False),
)
def _tokenize(value_hbm, erep_hbm, out_hbm, erep_v, vals_v, out_v):
    wid = lax.axis_index("s") * NC + lax.axis_index("c")
    base = wid * CHUNK
    with jax.named_scope("dma_in"):
        pltpu.sync_copy(erep_hbm, erep_v)
        pltpu.sync_copy(value_hbm.at[pl.ds(base, CHUNK)], vals_v)

    lane = lax.iota(jnp.int32, L)

    @plsc.parallel_loop(0, GROUPS, unroll=16)
    def group(g):
        v = vals_v[pl.ds(g * L, L)]
        c = lane
        for half in _HALVES:
            ev = plsc.load_gather(erep_v, [c + (half - 1) * L])
            c = c + jnp.where(ev < v, half * L, 0)
        ec = plsc.load_gather(erep_v, [c])
        i0 = c + jnp.where(ec < v, L, 0)
        ea = plsc.load_gather(erep_v, [jnp.maximum(i0 - L, lane)])
        eb = plsc.load_gather(erep_v, [jnp.minimum(i0, (K - 1) * L + lane)])
        ind = ((v - ea) > (eb - v)) & (i0 < K * L)
        dstar = jnp.where(ind, eb - v, v - ea)
        c2 = lane
        for half in _HALVES:
            ev = plsc.load_gather(erep_v, [c2 + (half - 1) * L])
            c2 = c2 + jnp.where((v - ev) > dstar, half * L, 0)
        ec2 = plsc.load_gather(erep_v, [c2])
        ans = c2 + jnp.where((v - ec2) > dstar, L, 0)
        out_v[pl.ds(g * L, L)] = jax.lax.shift_right_logical(ans, 4)

    pltpu.sync_copy(out_v, out_hbm.at[pl.ds(base, CHUNK)])


def kernel(value, embed):
    erep = jnp.repeat(embed, L)
    idx = _tokenize(value, erep)
    return idx[:, None]
